# baseline (speedup 1.0000x reference)
import jax
import jax.numpy as jnp
from jax.experimental import pallas as pl
from jax.experimental.pallas import tpu as pltpu

_K = 256
_ROWS = 131072
_C = 81
_R = 16384
_G = _ROWS // _R


def _body(d_ref, dl_ref, l_ref, out_ref, s_sc, e_sc):
    i = pl.program_id(0)
    x = d_ref[...]
    e = jnp.exp(x)
    lab = l_ref[...]
    onehot = jax.lax.broadcasted_iota(jnp.int32, (_C - 1, _R), 0) == lab
    me = jnp.where(onehot, e, 0.0)
    el = jnp.exp(dl_ref[...])
    ones = jnp.ones((1, _C - 1), jnp.float32)
    dn = (((1,), (0,)), ((), ()))
    s_row = jax.lax.dot_general(
        ones, e, dn, preferred_element_type=jnp.float32) + el
    e_row = jax.lax.dot_general(
        ones, me, dn, preferred_element_type=jnp.float32)
    e_row = jnp.where(lab == _C - 1, el, e_row)
    row = pl.multiple_of(i * 8, 8)
    s_sc[pl.ds(row, 1), :] = s_row
    e_sc[pl.ds(row, 1), :] = e_row

    @pl.when(i == _G - 1)
    def _():
        s = jnp.concatenate(
            [s_sc[pl.ds(8 * k, 1), :] for k in range(_G)], axis=0)
        ee = jnp.concatenate(
            [e_sc[pl.ds(8 * k, 1), :] for k in range(_G)], axis=0)
        r = jnp.maximum(s / ee, 1.0)
        bits = jax.lax.bitcast_convert_type(r, jnp.int32)

        def it(_, lohi):
            lo, hi = lohi
            mid = lo + ((hi - lo) >> 1)
            cnt = jnp.sum((bits >= mid).astype(jnp.int32))
            big = cnt >= _K
            return (jnp.where(big, mid, lo), jnp.where(big, hi, mid))

        lo, _ = jax.lax.fori_loop(
            0, 31, it, (jnp.int32(0), jnp.int32(0x7F800001)), unroll=False)

        ent = jnp.log(r)
        gt = bits > lo
        eq = bits == lo
        c_gt = jnp.sum(gt.astype(jnp.int32)).astype(jnp.float32)
        c_eq = jnp.sum(eq.astype(jnp.int32)).astype(jnp.float32)
        s_gt = jnp.sum(jnp.where(gt, ent, 0.0))
        s_eq = jnp.sum(jnp.where(eq, ent, 0.0))
        loss = (s_gt + (_K - c_gt) * (s_eq / c_eq)) / _K
        out_ref[0, 0] = loss


def kernel(dists, labels):
    dt = dists.T
    dlast = jax.lax.slice(dt, (_C - 1, 0), (_C, _ROWS))
    lab = labels.reshape(1, _ROWS).astype(jnp.int32)
    loss = pl.pallas_call(
        _body,
        grid=(_G,),
        in_specs=[
            pl.BlockSpec((_C - 1, _R), lambda i: (0, i)),
            pl.BlockSpec((1, _R), lambda i: (0, i)),
            pl.BlockSpec((1, _R), lambda i: (0, i)),
        ],
        out_specs=pl.BlockSpec(
            (1, 1), lambda i: (0, 0), memory_space=pltpu.MemorySpace.SMEM),
        out_shape=jax.ShapeDtypeStruct((1, 1), jnp.float32),
        scratch_shapes=[
            pltpu.VMEM((8 * _G, _R), jnp.float32),
            pltpu.VMEM((8 * _G, _R), jnp.float32),
        ],
    )(dt, dlast, lab)
    return loss[0, 0]


# speedup vs baseline: 1.0724x; 1.0724x over previous
import jax
import jax.numpy as jnp
from jax.experimental import pallas as pl
from jax.experimental.pallas import tpu as pltpu

_K = 256
_ROWS = 131072
_C = 81
_R = 16384
_G = _ROWS // _R


def _body(d_ref, l_ref, out_ref, s_sc, e_sc):
    i = pl.program_id(0)
    x = d_ref[...]
    e = jnp.exp(x)
    onehot = jax.lax.broadcasted_iota(jnp.int32, (_C, _R), 0) == l_ref[...]
    me = jnp.where(onehot, e, 0.0)
    ones = jnp.ones((1, _C), jnp.float32)
    dn = (((1,), (0,)), ((), ()))
    s_row = jax.lax.dot_general(
        ones, e, dn, preferred_element_type=jnp.float32)
    e_row = jax.lax.dot_general(
        ones, me, dn, preferred_element_type=jnp.float32)
    row = pl.multiple_of(i * 8, 8)
    s_sc[pl.ds(row, 1), :] = s_row
    e_sc[pl.ds(row, 1), :] = e_row

    @pl.when(i == _G - 1)
    def _():
        s = jnp.concatenate(
            [s_sc[pl.ds(8 * k, 1), :] for k in range(_G)], axis=0)
        ee = jnp.concatenate(
            [e_sc[pl.ds(8 * k, 1), :] for k in range(_G)], axis=0)
        r = jnp.maximum(s / ee, 1.0)
        bits = jax.lax.bitcast_convert_type(r, jnp.int32)

        def it(_, lohi):
            lo, hi = lohi
            mid = lo + ((hi - lo) >> 1)
            cnt = jnp.sum((bits >= mid).astype(jnp.int32))
            big = cnt >= _K
            return (jnp.where(big, mid, lo), jnp.where(big, hi, mid))

        lo, _ = jax.lax.fori_loop(
            0, 31, it, (jnp.int32(0), jnp.int32(0x7F800001)), unroll=False)

        ent = jnp.log(r)
        gt = bits > lo
        eq = bits == lo
        c_gt = jnp.sum(gt.astype(jnp.int32)).astype(jnp.float32)
        c_eq = jnp.sum(eq.astype(jnp.int32)).astype(jnp.float32)
        s_gt = jnp.sum(jnp.where(gt, ent, 0.0))
        s_eq = jnp.sum(jnp.where(eq, ent, 0.0))
        loss = (s_gt + (_K - c_gt) * (s_eq / c_eq)) / _K
        out_ref[0, 0] = loss


def kernel(dists, labels):
    dt = dists.T
    lab = labels.reshape(1, _ROWS).astype(jnp.int32)
    loss = pl.pallas_call(
        _body,
        grid=(_G,),
        in_specs=[
            pl.BlockSpec((_C, _R), lambda i: (0, i)),
            pl.BlockSpec((1, _R), lambda i: (0, i)),
        ],
        out_specs=pl.BlockSpec(
            (1, 1), lambda i: (0, 0), memory_space=pltpu.MemorySpace.SMEM),
        out_shape=jax.ShapeDtypeStruct((1, 1), jnp.float32),
        scratch_shapes=[
            pltpu.VMEM((8 * _G, _R), jnp.float32),
            pltpu.VMEM((8 * _G, _R), jnp.float32),
        ],
    )(dt, lab)
    return loss[0, 0]
